# baseline (speedup 1.0000x reference)
import functools

import jax
import jax.numpy as jnp
from jax import lax
from jax.experimental import pallas as pl
from jax.experimental.pallas import tpu as pltpu
from jax.experimental.pallas import tpu_sc as plsc

N = 10000
D = 128
E = 320000
NC, NS = 2, 16
NW = NC * NS
BLK = 128
BPT = (E + NW * BLK - 1) // (NW * BLK)
E_PAD = NW * BLK * BPT
EPT = E_PAD // NW
BPT_P = BPT + 1
BPT_G = BPT_P + 2
NPAD = 10240
RPT = NPAD // NW
RPS = NPAD // NS
ROWB = 1024

@functools.cache
def _build_sc_kernels():
    mesh = plsc.VectorSubcoreMesh(
        core_axis_name="c", subcore_axis_name="s",
        num_cores=NC, num_subcores=NS,
    )

    @functools.partial(
        pl.kernel,
        out_type=jax.ShapeDtypeStruct((NW, NPAD), jnp.float32),
        mesh=mesh,
        scratch_types=[
            pltpu.VMEM((NPAD,), jnp.float32),
            pltpu.VMEM((EPT,), jnp.int32),
        ],
        compiler_params=pltpu.CompilerParams(needs_layout_passes=False),
    )
    def sc_degree(dst_hbm, zflat_hbm, degp_hbm, degloc, dstall):
        wid = lax.axis_index("c") * NS + lax.axis_index("s")
        one16 = jnp.ones((16,), jnp.float32)

        pltpu.sync_copy(zflat_hbm, degloc)
        pltpu.sync_copy(dst_hbm.at[pl.ds(wid * EPT, EPT)], dstall)

        @pl.loop(0, EPT // 16)
        def _hist(i):
            idx = dstall[pl.ds(i * 16, 16)]
            plsc.addupdate_scatter(degloc, [idx], one16)

        pltpu.sync_copy(degloc, degp_hbm.at[wid])

    @functools.partial(
        pl.kernel,
        out_type=jax.ShapeDtypeStruct((NC * NPAD, D), jnp.float32),
        mesh=mesh,
        scratch_types=[
            pltpu.VMEM_SHARED((NPAD, D), jnp.float32),
            pltpu.VMEM((BLK, D), jnp.float32),
            pltpu.VMEM((BLK, D), jnp.float32),
            pltpu.VMEM((8, BLK), jnp.int32),
            pltpu.SemaphoreType.DMA,
            pltpu.SemaphoreType.DMA,
        ],
        compiler_params=pltpu.CompilerParams(needs_layout_passes=False),
    )
    def sc_scatter(hp_hbm, comb_hbm, zrows_hbm, parts_hbm,
                   acc, rows0, rows1, i8, sem0, sem1):
        c = lax.axis_index("c")
        s = lax.axis_index("s")
        wid = c * NS + s
        bufs = ((rows0, sem0), (rows1, sem1))

        pltpu.sync_copy(zrows_hbm, rows0)
        pltpu.sync_copy(rows0, acc.at[pl.ds(s * RPT, BLK)])
        pltpu.sync_copy(rows0, acc.at[pl.ds(s * RPT + BLK, BLK)])
        pltpu.sync_copy(rows0.at[pl.ds(0, RPT - 2 * BLK)],
                        acc.at[pl.ds(s * RPT + 2 * BLK, RPT - 2 * BLK)])
        plsc.subcore_barrier()

        @pl.loop(0, BPT_P // 4)
        def _g(g):
            pltpu.sync_copy(comb_hbm.at[wid, g], i8)
            for r in range(4):
                pltpu.async_copy(
                    hp_hbm.at[i8.at[2 * r]], rows0, sem0).wait()
                pltpu.sync_copy(rows0, acc.at[i8.at[2 * r + 1]], add=True)
        plsc.subcore_barrier()

        for k in range(RPS // BLK):
            r = s * RPS + k * BLK
            pltpu.sync_copy(acc.at[pl.ds(r, BLK)], rows0)
            pltpu.sync_copy(rows0, parts_hbm.at[pl.ds(c * NPAD + r, BLK)])

    return sc_degree, sc_scatter


def _prep_body(x_ref, w_ref, degp_ref, h_ref, dinv_ref):
    deg = jnp.sum(degp_ref[...], axis=0) + 1.0
    dinv = lax.rsqrt(deg)
    h = jnp.dot(x_ref[...], w_ref[...], preferred_element_type=jnp.float32)
    h_ref[...] = h * dinv[:, None]
    dinv_ref[...] = dinv[:, None]


def _mid_body(p_ref, hp_ref, dinv_ref, b_ref, w_ref, o_ref):
    s = p_ref[0] + p_ref[1] + hp_ref[...]
    pre = s * dinv_ref[...] + b_ref[...]
    h = jnp.maximum(pre, 0.0)
    o_ref[...] = (
        jnp.dot(h, w_ref[...], preferred_element_type=jnp.float32)
        * dinv_ref[...]
    )


def _fin_body(p_ref, hp_ref, dinv_ref, b_ref, o_ref):
    s = p_ref[0] + p_ref[1] + hp_ref[...]
    o_ref[...] = s * dinv_ref[...] + b_ref[...]


_GRID = (NPAD // ROWB,)
_row_spec = pl.BlockSpec((ROWB, D), lambda i: (i, 0))
_dinv_spec = pl.BlockSpec((ROWB, 1), lambda i: (i, 0))
_w_spec = pl.BlockSpec((D, D), lambda i: (0, 0))
_b_spec = pl.BlockSpec((1, D), lambda i: (0, 0))
_degp_spec = pl.BlockSpec((NW, ROWB), lambda i: (0, i))
_parts_spec = pl.BlockSpec((2, ROWB, D), lambda i: (0, i, 0))

_prep = pl.pallas_call(
    _prep_body,
    grid=_GRID,
    in_specs=[_row_spec, _w_spec, _degp_spec],
    out_specs=[_row_spec, _dinv_spec],
    out_shape=[
        jax.ShapeDtypeStruct((NPAD, D), jnp.float32),
        jax.ShapeDtypeStruct((NPAD, 1), jnp.float32),
    ],
)

_mid = pl.pallas_call(
    _mid_body,
    grid=_GRID,
    in_specs=[_parts_spec, _row_spec, _dinv_spec, _b_spec, _w_spec],
    out_specs=_row_spec,
    out_shape=jax.ShapeDtypeStruct((NPAD, D), jnp.float32),
)

_fin = pl.pallas_call(
    _fin_body,
    grid=_GRID,
    in_specs=[_parts_spec, _row_spec, _dinv_spec, _b_spec],
    out_specs=_row_spec,
    out_shape=jax.ShapeDtypeStruct((NPAD, D), jnp.float32),
)


@jax.jit
def kernel(x, edge_index, W1, b1, W2, b2):
    _sc_degree, _sc_scatter = _build_sc_kernels()
    src = edge_index[0].astype(jnp.int32)
    dst = edge_index[1].astype(jnp.int32)
    pad = E_PAD - E
    src_p = jnp.concatenate([src, jnp.zeros((pad,), jnp.int32)])
    dst_p = jnp.concatenate([dst, jnp.full((pad,), N, jnp.int32)])
    src4 = jnp.pad(src_p.reshape(NW, BPT, BLK), ((0, 0), (0, 1), (0, 0)))
    dst4 = jnp.pad(dst_p.reshape(NW, BPT, BLK), ((0, 0), (0, 1), (0, 0)),
                   constant_values=N)
    comb = jnp.stack([src4, dst4], axis=2).reshape(NW, BPT_P // 4, 8, BLK)
    zrows = jnp.zeros((BLK, D), jnp.float32)
    zflat = jnp.zeros((NPAD,), jnp.float32)
    b1r = b1.reshape(1, D)
    b2r = b2.reshape(1, D)
    x_p = jnp.pad(x, ((0, NPAD - N), (0, 0)))

    degp = _sc_degree(dst_p, zflat)
    h1p, dinv = _prep(x_p, W1, degp)
    parts1 = _sc_scatter(h1p, comb, zrows)
    parts1 = parts1.reshape(NC, NPAD, D)
    h2p = _mid(parts1, h1p, dinv, b1r, W2)
    parts2 = _sc_scatter(h2p, comb, zrows)
    parts2 = parts2.reshape(NC, NPAD, D)
    out = _fin(parts2, h2p, dinv, b2r)
    return out[:N]


# speedup vs baseline: 1.2488x; 1.2488x over previous
import functools

import jax
import jax.numpy as jnp
from jax import lax
from jax.experimental import pallas as pl
from jax.experimental.pallas import tpu as pltpu
from jax.experimental.pallas import tpu_sc as plsc

N = 10000
D = 128
E = 320000
NC, NS = 2, 16
NW = NC * NS
BLK = 128
BPT = (E + NW * BLK - 1) // (NW * BLK)
E_PAD = NW * BLK * BPT
EPT = E_PAD // NW
BPT_P = BPT + 1
BPT_G = BPT_P + 2
NPAD = 10240
RPT = NPAD // NW
RPS = NPAD // NS
ROWB = 1024

@functools.cache
def _build_sc_kernels():
    mesh = plsc.VectorSubcoreMesh(
        core_axis_name="c", subcore_axis_name="s",
        num_cores=NC, num_subcores=NS,
    )

    @functools.partial(
        pl.kernel,
        out_type=jax.ShapeDtypeStruct((NW, NPAD), jnp.float32),
        mesh=mesh,
        scratch_types=[
            pltpu.VMEM((NPAD,), jnp.float32),
            pltpu.VMEM((EPT,), jnp.int32),
        ],
        compiler_params=pltpu.CompilerParams(needs_layout_passes=False),
    )
    def sc_degree(dst_hbm, zflat_hbm, degp_hbm, degloc, dstall):
        wid = lax.axis_index("c") * NS + lax.axis_index("s")
        one16 = jnp.ones((16,), jnp.float32)

        pltpu.sync_copy(zflat_hbm, degloc)
        pltpu.sync_copy(dst_hbm.at[pl.ds(wid * EPT, EPT)], dstall)

        @pl.loop(0, EPT // 16)
        def _hist(i):
            idx = dstall[pl.ds(i * 16, 16)]
            plsc.addupdate_scatter(degloc, [idx], one16)

        pltpu.sync_copy(degloc, degp_hbm.at[wid])

    @functools.partial(
        pl.kernel,
        out_type=jax.ShapeDtypeStruct((NC * NPAD, D), jnp.float32),
        mesh=mesh,
        scratch_types=[
            pltpu.VMEM_SHARED((NPAD, D), jnp.float32),
            pltpu.VMEM((BLK, D), jnp.float32),
            pltpu.VMEM((BLK,), jnp.int32),
            pltpu.VMEM((BLK,), jnp.int32),
            pltpu.SemaphoreType.DMA,
        ],
        compiler_params=pltpu.CompilerParams(needs_layout_passes=False),
    )
    def sc_scatter(hp_hbm, src_hbm, dst_hbm, zrows_hbm, parts_hbm,
                   acc, rows0, srcbuf, dstbuf, sem0):
        c = lax.axis_index("c")
        s = lax.axis_index("s")
        wid = c * NS + s

        pltpu.sync_copy(zrows_hbm, rows0)
        pltpu.sync_copy(rows0, acc.at[pl.ds(s * RPT, BLK)])
        pltpu.sync_copy(rows0, acc.at[pl.ds(s * RPT + BLK, BLK)])
        pltpu.sync_copy(rows0.at[pl.ds(0, RPT - 2 * BLK)],
                        acc.at[pl.ds(s * RPT + 2 * BLK, RPT - 2 * BLK)])
        plsc.subcore_barrier()

        @pl.loop(0, BPT)
        def _blk(j):
            base = (wid * BPT + j) * BLK
            pltpu.sync_copy(src_hbm.at[pl.ds(base, BLK)], srcbuf)
            pltpu.sync_copy(dst_hbm.at[pl.ds(base, BLK)], dstbuf)
            pltpu.async_copy(hp_hbm.at[srcbuf], rows0, sem0).wait()
            pltpu.sync_copy(rows0, acc.at[dstbuf], add=True)
        plsc.subcore_barrier()

        for k in range(RPS // BLK):
            r = s * RPS + k * BLK
            pltpu.sync_copy(acc.at[pl.ds(r, BLK)], rows0)
            pltpu.sync_copy(rows0, parts_hbm.at[pl.ds(c * NPAD + r, BLK)])

    return sc_degree, sc_scatter


def _prep_body(x_ref, w_ref, degp_ref, h_ref, dinv_ref):
    deg = jnp.sum(degp_ref[...], axis=0) + 1.0
    dinv = lax.rsqrt(deg)
    h = jnp.dot(x_ref[...], w_ref[...], preferred_element_type=jnp.float32)
    h_ref[...] = h * dinv[:, None]
    dinv_ref[...] = dinv[:, None]


def _mid_body(p_ref, hp_ref, dinv_ref, b_ref, w_ref, o_ref):
    s = p_ref[0] + p_ref[1] + hp_ref[...]
    pre = s * dinv_ref[...] + b_ref[...]
    h = jnp.maximum(pre, 0.0)
    o_ref[...] = (
        jnp.dot(h, w_ref[...], preferred_element_type=jnp.float32)
        * dinv_ref[...]
    )


def _fin_body(p_ref, hp_ref, dinv_ref, b_ref, o_ref):
    s = p_ref[0] + p_ref[1] + hp_ref[...]
    o_ref[...] = s * dinv_ref[...] + b_ref[...]


_GRID = (NPAD // ROWB,)
_row_spec = pl.BlockSpec((ROWB, D), lambda i: (i, 0))
_dinv_spec = pl.BlockSpec((ROWB, 1), lambda i: (i, 0))
_w_spec = pl.BlockSpec((D, D), lambda i: (0, 0))
_b_spec = pl.BlockSpec((1, D), lambda i: (0, 0))
_degp_spec = pl.BlockSpec((NW, ROWB), lambda i: (0, i))
_parts_spec = pl.BlockSpec((2, ROWB, D), lambda i: (0, i, 0))

_prep = pl.pallas_call(
    _prep_body,
    grid=_GRID,
    in_specs=[_row_spec, _w_spec, _degp_spec],
    out_specs=[_row_spec, _dinv_spec],
    out_shape=[
        jax.ShapeDtypeStruct((NPAD, D), jnp.float32),
        jax.ShapeDtypeStruct((NPAD, 1), jnp.float32),
    ],
)

_mid = pl.pallas_call(
    _mid_body,
    grid=_GRID,
    in_specs=[_parts_spec, _row_spec, _dinv_spec, _b_spec, _w_spec],
    out_specs=_row_spec,
    out_shape=jax.ShapeDtypeStruct((NPAD, D), jnp.float32),
)

_fin = pl.pallas_call(
    _fin_body,
    grid=_GRID,
    in_specs=[_parts_spec, _row_spec, _dinv_spec, _b_spec],
    out_specs=_row_spec,
    out_shape=jax.ShapeDtypeStruct((NPAD, D), jnp.float32),
)


@jax.jit
def kernel(x, edge_index, W1, b1, W2, b2):
    _sc_degree, _sc_scatter = _build_sc_kernels()
    src = edge_index[0].astype(jnp.int32)
    dst = edge_index[1].astype(jnp.int32)
    pad = E_PAD - E
    src_p = jnp.concatenate([src, jnp.zeros((pad,), jnp.int32)])
    dst_p = jnp.concatenate([dst, jnp.full((pad,), N, jnp.int32)])
    zrows = jnp.zeros((BLK, D), jnp.float32)
    zflat = jnp.zeros((NPAD,), jnp.float32)
    b1r = b1.reshape(1, D)
    b2r = b2.reshape(1, D)
    x_p = jnp.pad(x, ((0, NPAD - N), (0, 0)))

    degp = _sc_degree(dst_p, zflat)
    h1p, dinv = _prep(x_p, W1, degp)
    parts1 = _sc_scatter(h1p, src_p, dst_p, zrows)
    parts1 = parts1.reshape(NC, NPAD, D)
    h2p = _mid(parts1, h1p, dinv, b1r, W2)
    parts2 = _sc_scatter(h2p, src_p, dst_p, zrows)
    parts2 = parts2.reshape(NC, NPAD, D)
    out = _fin(parts2, h2p, dinv, b2r)
    return out[:N]
